# baseline (speedup 1.0000x reference)
import functools

import jax
import jax.numpy as jnp
from jax import lax
from jax.experimental import pallas as pl
from jax.experimental.pallas import tpu as pltpu
from jax.experimental.pallas import tpu_sc as plsc

D = 64
SCALE = 8.0
BT = 128
CHA = 160


@functools.cache
def _sc_info():
    info = plsc.get_sparse_core_info()
    return info.num_cores, info.num_subcores


@functools.cache
def _make_pair_table(V: int):
    NC, NS = _sc_info()
    NW = NC * NS
    n_chunks = V // CHA
    rounds = (n_chunks + NW - 1) // NW
    mesh = plsc.VectorSubcoreMesh(core_axis_name="c", subcore_axis_name="s")

    @functools.partial(
        pl.kernel,
        mesh=mesh,
        compiler_params=pltpu.CompilerParams(needs_layout_passes=False),
        out_type=jax.ShapeDtypeStruct((V // 2, 128), jnp.float32),
        scratch_types=[
            pltpu.VMEM((2, CHA, D), jnp.float32),
            pltpu.VMEM((2, CHA // 2, 2 * D), jnp.float32),
            pltpu.SemaphoreType.DMA,
            pltpu.SemaphoreType.DMA,
            pltpu.SemaphoreType.DMA,
            pltpu.SemaphoreType.DMA,
        ],
    )
    def pair_kernel(tab_hbm, pair_hbm, in_v, out_v, gi0, gi1, go0, go1):
        wid = lax.axis_index("s") * NC + lax.axis_index("c")
        gsem = (gi0, gi1)
        osem = (go0, go1)

        def chunk_id(j):
            return wid + j * NW

        def in_start(j, buf):
            pltpu.async_copy(tab_hbm.at[pl.ds(chunk_id(j) * CHA, CHA), :],
                             in_v.at[buf], gsem[buf])

        def in_wait(j, buf):
            pltpu.make_async_copy(tab_hbm.at[pl.ds(chunk_id(j) * CHA, CHA), :],
                                  in_v.at[buf], gsem[buf]).wait()

        def out_start(j, buf):
            pltpu.async_copy(
                out_v.at[buf],
                pair_hbm.at[pl.ds(chunk_id(j) * (CHA // 2), CHA // 2), :],
                osem[buf])

        def out_wait(j, buf):
            pltpu.make_async_copy(
                out_v.at[buf],
                pair_hbm.at[pl.ds(chunk_id(j) * (CHA // 2), CHA // 2), :],
                osem[buf]).wait()

        def repack(buf):
            def p_body(p, carry):
                vs = []
                for half in range(2):
                    for jj in range(D // 16):
                        vs.append(in_v[buf, 2 * p + half, pl.ds(jj * 16, 16)])
                for half in range(2):
                    for jj in range(D // 16):
                        out_v[buf, p, pl.ds(half * D + jj * 16, 16)] = (
                            vs[half * (D // 16) + jj] * SCALE)
                return carry

            lax.fori_loop(0, CHA // 2, p_body, 0, unroll=2)

        def guarded(j, fn):
            @pl.when(chunk_id(j) < n_chunks)
            def _():
                fn()

        in_start(0, 0)

        def round_body(jj, carry):
            j0 = jj * 2
            guarded(j0 + 1, lambda: in_start(j0 + 1, 1))
            in_wait(j0, 0)

            @pl.when(jj > 0)
            def _():
                out_wait(j0 - 2, 0)

            repack(0)
            out_start(j0, 0)
            guarded(j0 + 2, lambda: in_start(j0 + 2, 0))
            guarded(j0 + 1, lambda: in_wait(j0 + 1, 1))

            @pl.when(jj > 0)
            def _():
                guarded(j0 - 1, lambda: out_wait(j0 - 1, 1))

            guarded(j0 + 1, lambda: repack(1))
            guarded(j0 + 1, lambda: out_start(j0 + 1, 1))
            return carry

        lax.fori_loop(0, (rounds + 1) // 2, round_body, 0)
        out_wait_last = rounds * NW
        guarded(2 * ((rounds + 1) // 2) - 2, lambda: out_wait(
            2 * ((rounds + 1) // 2) - 2, 0))
        guarded(2 * ((rounds + 1) // 2) - 1, lambda: out_wait(
            2 * ((rounds + 1) // 2) - 1, 1))

    return pair_kernel


@functools.cache
def _make_sc_embed(N: int, B: int):
    NC, NS = _sc_info()
    NW = NC * NS
    assert B == BT * NW and N % 2 == 0
    mesh = plsc.VectorSubcoreMesh(core_axis_name="c", subcore_axis_name="s")

    @functools.partial(
        pl.kernel,
        mesh=mesh,
        compiler_params=pltpu.CompilerParams(needs_layout_passes=False),
        out_type=jax.ShapeDtypeStruct((N, D, B), jnp.float32),
        scratch_types=[
            pltpu.VMEM((N, BT), jnp.int32),
            pltpu.VMEM((N, BT), jnp.int32),
            pltpu.VMEM((2, BT, 128), jnp.float32),
            pltpu.VMEM((2, D, BT), jnp.float32),
            pltpu.SemaphoreType.DMA,
            pltpu.SemaphoreType.DMA,
            pltpu.SemaphoreType.DMA,
            pltpu.SemaphoreType.DMA,
        ],
    )
    def sc_embed(xT_hbm, tab2_hbm, out_hbm, idx_v, pair_v, rows_v, out_v,
                 g0, g1, o0, o1):
        wid = lax.axis_index("s") * NC + lax.axis_index("c")
        b0 = wid * BT

        pltpu.sync_copy(xT_hbm.at[:, pl.ds(b0, BT)], idx_v)

        def pair_body(n, carry):
            for g in range(BT // 16):
                sl = pl.ds(g * 16, 16)
                pair_v[n, sl] = lax.shift_right_logical(idx_v[n, sl], 1)
            return carry

        lax.fori_loop(0, N, pair_body, 0)

        row_ids = [jnp.arange(bg * 16, bg * 16 + 16, dtype=jnp.int32)
                   for bg in range(8)]
        gsem = (g0, g1)
        osem = (o0, o1)

        def gather_start(n, buf):
            pltpu.async_copy(tab2_hbm.at[pair_v.at[n]], rows_v.at[buf],
                             gsem[buf])

        def gather_wait(n, buf):
            pltpu.make_async_copy(tab2_hbm.at[pair_v.at[n]], rows_v.at[buf],
                                  gsem[buf]).wait()

        def out_start(n, buf):
            pltpu.async_copy(out_v.at[buf], out_hbm.at[n, :, pl.ds(b0, BT)],
                             osem[buf])

        def out_wait(n, buf):
            pltpu.make_async_copy(out_v.at[buf], out_hbm.at[n, :, pl.ds(b0, BT)],
                                  osem[buf]).wait()

        lane = jnp.arange(16, dtype=jnp.int32)

        def transpose_item(n, buf):
            cols0 = []
            for bg in range(8):
                xv = idx_v[n, pl.ds(bg * 16, 16)]
                cols0.append((xv & 1) << 6)

            def d_body(t, cols):
                dpv = (t + lane) & (D - 1)
                vs = [plsc.load_gather(rows_v.at[buf],
                                      [row_ids[bg], cols[bg] + dpv])
                      for bg in range(8)]
                for bg in range(8):
                    plsc.store_scatter(out_v.at[buf], [dpv, row_ids[bg]],
                                       vs[bg])
                return cols

            lax.fori_loop(0, D, d_body, tuple(cols0), unroll=2)

        gather_start(0, 0)

        def loop_body(kk, carry):
            n0 = kk * 2
            gather_start(n0 + 1, 1)
            gather_wait(n0, 0)

            @pl.when(kk > 0)
            def _():
                out_wait(n0 - 2, 0)

            transpose_item(n0, 0)
            out_start(n0, 0)

            @pl.when(kk < N // 2 - 1)
            def _():
                gather_start(n0 + 2, 0)

            gather_wait(n0 + 1, 1)

            @pl.when(kk > 0)
            def _():
                out_wait(n0 - 1, 1)

            transpose_item(n0 + 1, 1)
            out_start(n0 + 1, 1)
            return carry

        lax.fori_loop(0, N // 2, loop_body, 0)
        out_wait(N - 2, 0)
        out_wait(N - 1, 1)

    return sc_embed


def kernel(x, table):
    B_, N_ = x.shape
    V = table.shape[0]
    xT = x.astype(jnp.int32).T
    tab2 = _make_pair_table(V)(table)
    out_t = _make_sc_embed(N_, B_)(xT, tab2)
    return out_t.transpose(2, 0, 1)


# speedup vs baseline: 1.6334x; 1.6334x over previous
import functools

import jax
import jax.numpy as jnp
from jax import lax
from jax.experimental import pallas as pl
from jax.experimental.pallas import tpu as pltpu
from jax.experimental.pallas import tpu_sc as plsc

D = 64
SCALE = 8.0
BT = 128
CHA = 160


@functools.cache
def _sc_info():
    info = plsc.get_sparse_core_info()
    return info.num_cores, info.num_subcores


@functools.cache
def _make_pair_table(V: int):
    NC, NS = _sc_info()
    NW = NC * NS
    n_full = V // 128
    tail = V - n_full * 128
    rounds = (n_full + NW - 1) // NW
    mesh = plsc.VectorSubcoreMesh(core_axis_name="c", subcore_axis_name="s")

    @functools.partial(
        pl.kernel,
        mesh=mesh,
        compiler_params=pltpu.CompilerParams(needs_layout_passes=False),
        out_type=jax.ShapeDtypeStruct((V // 2, 128), jnp.float32),
        scratch_types=[
            pltpu.VMEM((2, D, 128), jnp.float32),
            pltpu.VMEM((2, D, 2 * D), jnp.float32),
            pltpu.SemaphoreType.DMA,
            pltpu.SemaphoreType.DMA,
            pltpu.SemaphoreType.DMA,
            pltpu.SemaphoreType.DMA,
        ],
    )
    def pair_kernel(tabT_hbm, tail_hbm, pair_hbm, in_v, out_v, gi0, gi1, go0, go1):
        wid = lax.axis_index("s") * NC + lax.axis_index("c")
        gsem = (gi0, gi1)
        osem = (go0, go1)
        lane = jnp.arange(16, dtype=jnp.int32)

        def chunk_id(j):
            return wid + j * NW

        def in_start(j, buf):
            pltpu.async_copy(tabT_hbm.at[:, pl.ds(chunk_id(j) * 128, 128)],
                             in_v.at[buf], gsem[buf])

        def in_wait(j, buf):
            pltpu.make_async_copy(tabT_hbm.at[:, pl.ds(chunk_id(j) * 128, 128)],
                                  in_v.at[buf], gsem[buf]).wait()

        def out_start(j, buf):
            pltpu.async_copy(out_v.at[buf],
                             pair_hbm.at[pl.ds(chunk_id(j) * 64, 64), :],
                             osem[buf])

        def out_wait(j, buf):
            pltpu.make_async_copy(out_v.at[buf],
                                  pair_hbm.at[pl.ds(chunk_id(j) * 64, 64), :],
                                  osem[buf]).wait()

        def repack(buf, npg):
            colc = [[2 * lane + (32 * pg + h) for h in range(2)]
                    for pg in range(npg)]

            def t_body(t, carry):
                dpv = (t + lane) & 15
                rows = [16 * dg + dpv for dg in range(4)]
                vs = []
                for pg in range(npg):
                    for h in range(2):
                        for dg in range(4):
                            vs.append(plsc.load_gather(
                                in_v.at[buf], [rows[dg], colc[pg][h]]))
                k = 0
                for pg in range(npg):
                    for h in range(2):
                        for dg in range(4):
                            plsc.store_scatter(
                                out_v.at[buf],
                                [16 * pg + lane, 64 * h + 16 * dg + dpv],
                                vs[k] * SCALE)
                            k += 1
                return carry

            lax.fori_loop(0, 16, t_body, 0)

        def guarded(j, fn):
            @pl.when(chunk_id(j) < n_full)
            def _():
                fn()

        in_start(0, 0)

        def round_body(jj, carry):
            j0 = jj * 2
            guarded(j0 + 1, lambda: in_start(j0 + 1, 1))
            guarded(j0, lambda: in_wait(j0, 0))

            @pl.when(jj > 0)
            def _():
                guarded(j0 - 2, lambda: out_wait(j0 - 2, 0))

            guarded(j0, lambda: repack(0, 4))
            guarded(j0, lambda: out_start(j0, 0))
            guarded(j0 + 2, lambda: in_start(j0 + 2, 0))
            guarded(j0 + 1, lambda: in_wait(j0 + 1, 1))

            @pl.when(jj > 0)
            def _():
                guarded(j0 - 1, lambda: out_wait(j0 - 1, 1))

            guarded(j0 + 1, lambda: repack(1, 4))
            guarded(j0 + 1, lambda: out_start(j0 + 1, 1))
            return carry

        half_rounds = (rounds + 1) // 2
        lax.fori_loop(0, half_rounds, round_body, 0)
        guarded(2 * half_rounds - 2, lambda: out_wait(2 * half_rounds - 2, 0))
        guarded(2 * half_rounds - 1, lambda: out_wait(2 * half_rounds - 1, 1))

        if tail:
            @pl.when(wid == 0)
            def _():
                pltpu.sync_copy(tail_hbm, in_v.at[0, pl.ds(0, tail // 2), :])
                pltpu.sync_copy(in_v.at[0, pl.ds(0, tail // 2), :],
                                pair_hbm.at[pl.ds(n_full * 64, tail // 2), :])

    return pair_kernel


@functools.cache
def _make_sc_embed(N: int, B: int):
    NC, NS = _sc_info()
    NW = NC * NS
    assert B == BT * NW and N % 2 == 0
    mesh = plsc.VectorSubcoreMesh(core_axis_name="c", subcore_axis_name="s")

    @functools.partial(
        pl.kernel,
        mesh=mesh,
        compiler_params=pltpu.CompilerParams(needs_layout_passes=False),
        out_type=jax.ShapeDtypeStruct((N, D, B), jnp.float32),
        scratch_types=[
            pltpu.VMEM((N, BT), jnp.int32),
            pltpu.VMEM((N, BT), jnp.int32),
            pltpu.VMEM((2, BT, 128), jnp.float32),
            pltpu.VMEM((2, D, BT), jnp.float32),
            pltpu.SemaphoreType.DMA,
            pltpu.SemaphoreType.DMA,
            pltpu.SemaphoreType.DMA,
            pltpu.SemaphoreType.DMA,
        ],
    )
    def sc_embed(xT_hbm, tab2_hbm, out_hbm, idx_v, pair_v, rows_v, out_v,
                 g0, g1, o0, o1):
        wid = lax.axis_index("s") * NC + lax.axis_index("c")
        b0 = wid * BT

        pltpu.sync_copy(xT_hbm.at[:, pl.ds(b0, BT)], idx_v)

        def pair_body(n, carry):
            for g in range(BT // 16):
                sl = pl.ds(g * 16, 16)
                pair_v[n, sl] = lax.shift_right_logical(idx_v[n, sl], 1)
            return carry

        lax.fori_loop(0, N, pair_body, 0)

        row_ids = [jnp.arange(bg * 16, bg * 16 + 16, dtype=jnp.int32)
                   for bg in range(8)]
        gsem = (g0, g1)
        osem = (o0, o1)

        def gather_start(n, buf):
            pltpu.async_copy(tab2_hbm.at[pair_v.at[n]], rows_v.at[buf],
                             gsem[buf])

        def gather_wait(n, buf):
            pltpu.make_async_copy(tab2_hbm.at[pair_v.at[n]], rows_v.at[buf],
                                  gsem[buf]).wait()

        def out_start(n, buf):
            pltpu.async_copy(out_v.at[buf], out_hbm.at[n, :, pl.ds(b0, BT)],
                             osem[buf])

        def out_wait(n, buf):
            pltpu.make_async_copy(out_v.at[buf], out_hbm.at[n, :, pl.ds(b0, BT)],
                                  osem[buf]).wait()

        lane = jnp.arange(16, dtype=jnp.int32)

        def transpose_item(n, buf):
            cols0 = []
            for bg in range(8):
                xv = idx_v[n, pl.ds(bg * 16, 16)]
                cols0.append((xv & 1) << 6)

            def d_body(t, cols):
                dpv = (t + lane) & (D - 1)
                vs = [plsc.load_gather(rows_v.at[buf],
                                      [row_ids[bg], cols[bg] + dpv])
                      for bg in range(8)]
                for bg in range(8):
                    plsc.store_scatter(out_v.at[buf], [dpv, row_ids[bg]],
                                       vs[bg])
                return cols

            lax.fori_loop(0, D, d_body, tuple(cols0), unroll=2)

        gather_start(0, 0)

        def loop_body(kk, carry):
            n0 = kk * 2
            gather_start(n0 + 1, 1)
            gather_wait(n0, 0)

            @pl.when(kk > 0)
            def _():
                out_wait(n0 - 2, 0)

            transpose_item(n0, 0)
            out_start(n0, 0)

            @pl.when(kk < N // 2 - 1)
            def _():
                gather_start(n0 + 2, 0)

            gather_wait(n0 + 1, 1)

            @pl.when(kk > 0)
            def _():
                out_wait(n0 - 1, 1)

            transpose_item(n0 + 1, 1)
            out_start(n0 + 1, 1)
            return carry

        lax.fori_loop(0, N // 2, loop_body, 0)
        out_wait(N - 2, 0)
        out_wait(N - 1, 1)

    return sc_embed


def kernel(x, table):
    B_, N_ = x.shape
    V = table.shape[0]
    xT = x.astype(jnp.int32).T
    n_full = (V // 128) * 128
    tailP = table[n_full:].reshape((V - n_full) // 2, 128) * SCALE
    tab2 = _make_pair_table(V)(table.T, tailP)
    out_t = _make_sc_embed(N_, B_)(xT, tab2)
    return out_t.transpose(2, 0, 1)
